# baseline (speedup 1.0000x reference)
import functools

import jax
import jax.numpy as jnp
from jax import lax
from jax.experimental import pallas as pl
from jax.experimental.pallas import tpu as pltpu
from jax.experimental.pallas import tpu_sc as plsc

INP = 1000000
EMBED_DIM = 32
BATCH = 16384
FIELDS = 26
B = BATCH * FIELDS

NC, NS = 2, 16
NW = NC * NS
BPW = B // NW
CH = 1664
NCH = BPW // CH
COLS = BATCH // NW

_mesh = plsc.VectorSubcoreMesh(core_axis_name="c", subcore_axis_name="s")


@functools.partial(
    pl.kernel,
    out_type=jax.ShapeDtypeStruct((B,), jnp.int32),
    mesh=_mesh,
    scratch_types=[
        pltpu.VMEM((FIELDS, COLS), jnp.int32),
        pltpu.SemaphoreType.DMA,
    ],
    compiler_params=pltpu.CompilerParams(
        use_tc_tiling_on_sc=True, needs_layout_passes=False),
)
def _fmt(xt_hbm, out_hbm, buf, sem):
    wid = lax.axis_index("s") * NC + lax.axis_index("c")
    c0 = wid * COLS
    pltpu.sync_copy(xt_hbm.at[:, pl.ds(c0, COLS)], buf)
    for f in range(FIELDS):
        pltpu.async_copy(
            buf.at[f], out_hbm.at[pl.ds(f * BATCH + c0, COLS)], sem)
    for f in range(FIELDS):
        pltpu.make_async_copy(
            buf.at[f], out_hbm.at[pl.ds(f * BATCH + c0, COLS)], sem).wait()


NBLK = INP // 128


@functools.partial(
    pl.kernel,
    out_type=jax.ShapeDtypeStruct((INP * EMBED_DIM,), jnp.float32),
    mesh=_mesh,
    scratch_types=[
        pltpu.VMEM((EMBED_DIM, 128), jnp.float32),
        pltpu.VMEM((EMBED_DIM, 128), jnp.float32),
        pltpu.VMEM((4096,), jnp.float32),
        pltpu.VMEM((4096,), jnp.float32),
        pltpu.VMEM((32, 16), jnp.int32),
        pltpu.VMEM((32, 16), jnp.int32),
        pltpu.SemaphoreType.DMA,
        pltpu.SemaphoreType.DMA,
        pltpu.SemaphoreType.DMA,
        pltpu.SemaphoreType.DMA,
    ],
    compiler_params=pltpu.CompilerParams(
        use_tc_tiling_on_sc=True, needs_layout_passes=False),
)
def _tpose(tt_hbm, tail_hbm, out_hbm, buf0, buf1, dbuf0, dbuf1, mtab, dtab,
           si0, si1, so0, so1):
    wid = lax.axis_index("s") * NC + lax.axis_index("c")
    iota = lax.iota(jnp.int32, 16)
    for k in range(32):
        m = (k + iota) & 31
        mtab[k, :] = m
        dtab[k, :] = iota * 32 + m
    lvecs = [l0 * 16 + iota for l0 in range(8)]
    bufs = (buf0, buf1)
    dbufs = (dbuf0, dbuf1)
    si = (si0, si1)
    so = (so0, so1)
    npw = NBLK // NW
    lo = wid * npw

    def in_copy(c, b):
        return [pltpu.make_async_copy(
            tt_hbm.at[pl.ds(e4 * 8, 8), pl.ds(c * 128, 128)],
            bufs[b].at[pl.ds(e4 * 8, 8)], si[b]) for e4 in range(4)]

    def out_copy(c, b):
        return pltpu.make_async_copy(
            dbufs[b], out_hbm.at[pl.ds(c * 4096, 4096)], so[b])

    def transpose_block(b, ncol16):
        def kbody(k, _):
            m = mtab[k, :]
            d = dtab[k, :]
            for l0 in range(ncol16):
                v = plsc.load_gather(bufs[b], [m, lvecs[l0]])
                plsc.store_scatter(dbufs[b], [d + l0 * 512], v)
            return 0
        lax.fori_loop(0, 32, kbody, 0)

    for cp in in_copy(lo, 0):
        cp.start()

    def body(j, _):
        for b in range(2):
            c = lo + 2 * j + b
            for cp in in_copy(c, b):
                cp.wait()

            @pl.when(2 * j + b + 1 < npw)
            def _():
                for cp in in_copy(c + 1, 1 - b):
                    cp.start()
            @pl.when(2 * j + b >= 2)
            def _():
                out_copy(c - 2, b).wait()
            transpose_block(b, 8)
            out_copy(c, b).start()
        return 0

    lax.fori_loop(0, npw // 2, body, 0)
    out_copy(lo + npw - 2, (npw - 2) % 2).wait()
    out_copy(lo + npw - 1, (npw - 1) % 2).wait()

    @pl.when(wid >= NW - 4)
    def _():
        c = NW * npw + (wid - (NW - 4))
        for cp in in_copy(c, 0):
            cp.start()
        for cp in in_copy(c, 0):
            cp.wait()
        transpose_block(0, 8)
        out_copy(c, 0).start()
        out_copy(c, 0).wait()

    @pl.when(wid == NW - 1)
    def _():
        pltpu.sync_copy(tail_hbm, dbuf0.at[pl.ds(0, 2048)])
        pltpu.sync_copy(dbuf0.at[pl.ds(0, 2048)],
                        out_hbm.at[pl.ds(NBLK * 4096, 2048)])


G = 1024
NG = BPW // G
GPF = BATCH // (G // 128 * 128)
OUTW = FIELDS * BATCH * EMBED_DIM


@functools.partial(
    pl.kernel,
    out_type=jax.ShapeDtypeStruct((OUTW,), jnp.float32),
    mesh=_mesh,
    scratch_types=[
        pltpu.VMEM((NG, G), jnp.int32),
        pltpu.VMEM((G, EMBED_DIM), jnp.float32),
        pltpu.VMEM((G, EMBED_DIM), jnp.float32),
        pltpu.VMEM((32 * G,), jnp.float32),
        pltpu.VMEM((32, 16), jnp.int32),
        pltpu.VMEM((32, 16), jnp.int32),
        pltpu.VMEM((64, 16), jnp.int32),
        pltpu.SemaphoreType.DMA,
        pltpu.SemaphoreType.DMA,
        pltpu.SemaphoreType.DMA,
        pltpu.SemaphoreType.DMA,
        pltpu.SemaphoreType.DMA,
    ],
    compiler_params=pltpu.CompilerParams(
        use_tc_tiling_on_sc=False, needs_layout_passes=False),
)
def _embed(table_hbm, idx_hbm, out_hbm, idx_v, rows0, rows1, dbuf,
           mtab, ftab, jtab, si, sg0, sg1, so0, so1):
    wid = lax.axis_index("s") * NC + lax.axis_index("c")
    iota = lax.iota(jnp.int32, 16)
    for k in range(32):
        m = (k + iota) & 31
        mtab[k, :] = m
        ftab[k, :] = (m >> 3) * 8192 + (m & 7) * 128 + iota
    for bc_l in range(8):
        for l0 in range(8):
            jtab[bc_l * 8 + l0, :] = bc_l * 128 + l0 * 16 + iota
    rows = (rows0, rows1)
    sg = (sg0, sg1)
    so = (so0, so1)
    pltpu.sync_copy(idx_hbm.at[wid], idx_v)

    def gather_copy(g, b):
        return pltpu.make_async_copy(
            table_hbm.at[idx_v.at[g]], rows[b], sg[b])

    def out_copies(g, b):
        gid = wid * NG + g
        base = (gid // GPF) * 524288 + (gid % GPF) * 8192
        return [pltpu.make_async_copy(
            dbuf.at[pl.ds(e4 * 8192, 8192)],
            out_hbm.at[pl.ds(base + e4 * 131072, 8192)], so[b])
            for e4 in range(4)]

    def transpose(b):
        def kbody(k, _):
            m = mtab[k, :]
            ft = ftab[k, :]
            for bc_l in range(8):
                for l0 in range(8):
                    jv = jtab[bc_l * 8 + l0, :]
                    v = plsc.load_gather(rows[b], [jv, m])
                    plsc.store_scatter(dbuf, [ft + (bc_l * 1024 + l0 * 16)], v)
            return 0
        lax.fori_loop(0, 32, kbody, 0)

    def step(g, b):
        gather_copy(g, b).wait()

        @pl.when(g + 1 < NG)
        def _():
            gather_copy(g + 1, 1 - b).start()
        @pl.when(g >= 1)
        def _():
            for cp in out_copies(g - 1, 1 - b):
                cp.wait()
        transpose(b)
        for cp in out_copies(g, b):
            cp.start()

    gather_copy(0, 0).start()

    def body(j, _):
        step(2 * j, 0)
        step(2 * j + 1, 1)
        return 0

    lax.fori_loop(0, NG // 2, body, 0)
    step(NG - 1, 0)
    for cp in out_copies(NG - 1, 0):
        cp.wait()


def kernel(x, table):
    tail = table[INP - 64:].reshape(-1)
    tlin = _tpose(table.T, tail)
    tbl = tlin.reshape(INP, EMBED_DIM)
    idx = _fmt(x.T.astype(jnp.int32)).reshape(NW, NG, G)
    out = _embed(tbl, idx)
    out = out.reshape(FIELDS, 4, 128, 8, 128).transpose(2, 4, 0, 1, 3)
    return out.reshape(BATCH, FIELDS, EMBED_DIM)


# speedup vs baseline: 1.8112x; 1.8112x over previous
import functools

import jax
import jax.numpy as jnp
from jax import lax
from jax.experimental import pallas as pl
from jax.experimental.pallas import tpu as pltpu
from jax.experimental.pallas import tpu_sc as plsc

INP = 1000000
EMBED_DIM = 32
BATCH = 16384
FIELDS = 26
B = BATCH * FIELDS

NC, NS = 2, 16
NW = NC * NS
BPW = B // NW
CH = 1664
NCH = BPW // CH
COLS = BATCH // NW

_mesh = plsc.VectorSubcoreMesh(core_axis_name="c", subcore_axis_name="s")


@functools.partial(
    pl.kernel,
    out_type=jax.ShapeDtypeStruct((B,), jnp.int32),
    mesh=_mesh,
    scratch_types=[
        pltpu.VMEM((FIELDS, COLS), jnp.int32),
        pltpu.SemaphoreType.DMA,
    ],
    compiler_params=pltpu.CompilerParams(
        use_tc_tiling_on_sc=True, needs_layout_passes=False),
)
def _fmt(xt_hbm, out_hbm, buf, sem):
    wid = lax.axis_index("s") * NC + lax.axis_index("c")
    c0 = wid * COLS
    pltpu.sync_copy(xt_hbm.at[:, pl.ds(c0, COLS)], buf)
    for f in range(FIELDS):
        pltpu.async_copy(
            buf.at[f], out_hbm.at[pl.ds(f * BATCH + c0, COLS)], sem)
    for f in range(FIELDS):
        pltpu.make_async_copy(
            buf.at[f], out_hbm.at[pl.ds(f * BATCH + c0, COLS)], sem).wait()


NBLK = INP // 128


@functools.partial(
    pl.kernel,
    out_type=jax.ShapeDtypeStruct((INP * EMBED_DIM,), jnp.float32),
    mesh=_mesh,
    scratch_types=[
        pltpu.VMEM((EMBED_DIM, 128), jnp.float32),
        pltpu.VMEM((EMBED_DIM, 128), jnp.float32),
        pltpu.VMEM((4096,), jnp.float32),
        pltpu.VMEM((4096,), jnp.float32),
        pltpu.VMEM((32, 16), jnp.int32),
        pltpu.VMEM((32, 16), jnp.int32),
        pltpu.SemaphoreType.DMA,
        pltpu.SemaphoreType.DMA,
        pltpu.SemaphoreType.DMA,
        pltpu.SemaphoreType.DMA,
    ],
    compiler_params=pltpu.CompilerParams(
        use_tc_tiling_on_sc=True, needs_layout_passes=False),
)
def _tpose(tt_hbm, tail_hbm, out_hbm, buf0, buf1, dbuf0, dbuf1, mtab, dtab,
           si0, si1, so0, so1):
    wid = lax.axis_index("s") * NC + lax.axis_index("c")
    iota = lax.iota(jnp.int32, 16)
    for k in range(32):
        m = (k + iota) & 31
        mtab[k, :] = m
        dtab[k, :] = iota * 32 + m
    lvecs = [l0 * 16 + iota for l0 in range(8)]
    bufs = (buf0, buf1)
    dbufs = (dbuf0, dbuf1)
    si = (si0, si1)
    so = (so0, so1)
    npw = NBLK // NW
    lo = wid * npw

    def in_copy(c, b):
        return [pltpu.make_async_copy(
            tt_hbm.at[pl.ds(e4 * 8, 8), pl.ds(c * 128, 128)],
            bufs[b].at[pl.ds(e4 * 8, 8)], si[b]) for e4 in range(4)]

    def out_copy(c, b):
        return pltpu.make_async_copy(
            dbufs[b], out_hbm.at[pl.ds(c * 4096, 4096)], so[b])

    def transpose_block(b, ncol16):
        @plsc.parallel_loop(0, 32, unroll=1)
        def kbody(k):
            m = mtab[k, :]
            d = dtab[k, :]
            for l0 in range(ncol16):
                v = plsc.load_gather(bufs[b], [m, lvecs[l0]])
                plsc.store_scatter(dbufs[b], [d + l0 * 512], v)

    for cp in in_copy(lo, 0):
        cp.start()

    def body(j, _):
        for b in range(2):
            c = lo + 2 * j + b
            for cp in in_copy(c, b):
                cp.wait()

            @pl.when(2 * j + b + 1 < npw)
            def _():
                for cp in in_copy(c + 1, 1 - b):
                    cp.start()
            @pl.when(2 * j + b >= 2)
            def _():
                out_copy(c - 2, b).wait()
            transpose_block(b, 8)
            out_copy(c, b).start()
        return 0

    lax.fori_loop(0, npw // 2, body, 0)
    out_copy(lo + npw - 2, (npw - 2) % 2).wait()
    out_copy(lo + npw - 1, (npw - 1) % 2).wait()

    @pl.when(wid >= NW - 4)
    def _():
        c = NW * npw + (wid - (NW - 4))
        for cp in in_copy(c, 0):
            cp.start()
        for cp in in_copy(c, 0):
            cp.wait()
        transpose_block(0, 8)
        out_copy(c, 0).start()
        out_copy(c, 0).wait()

    @pl.when(wid == NW - 1)
    def _():
        pltpu.sync_copy(tail_hbm, dbuf0.at[pl.ds(0, 2048)])
        pltpu.sync_copy(dbuf0.at[pl.ds(0, 2048)],
                        out_hbm.at[pl.ds(NBLK * 4096, 2048)])


G = 1024
NG = BPW // G
GPF = BATCH // (G // 128 * 128)
OUTW = FIELDS * BATCH * EMBED_DIM


@functools.partial(
    pl.kernel,
    out_type=jax.ShapeDtypeStruct((OUTW,), jnp.float32),
    mesh=_mesh,
    scratch_types=[
        pltpu.VMEM((NG, G), jnp.int32),
        pltpu.VMEM((G, EMBED_DIM), jnp.float32),
        pltpu.VMEM((G, EMBED_DIM), jnp.float32),
        pltpu.VMEM((32 * G,), jnp.float32),
        pltpu.VMEM((32, 16), jnp.int32),
        pltpu.VMEM((32, 16), jnp.int32),
        pltpu.VMEM((64, 16), jnp.int32),
        pltpu.SemaphoreType.DMA,
        pltpu.SemaphoreType.DMA,
        pltpu.SemaphoreType.DMA,
        pltpu.SemaphoreType.DMA,
        pltpu.SemaphoreType.DMA,
    ],
    compiler_params=pltpu.CompilerParams(
        use_tc_tiling_on_sc=False, needs_layout_passes=False),
)
def _embed(table_hbm, idx_hbm, out_hbm, idx_v, rows0, rows1, dbuf,
           mtab, ftab, jtab, si, sg0, sg1, so0, so1):
    wid = lax.axis_index("s") * NC + lax.axis_index("c")
    iota = lax.iota(jnp.int32, 16)
    for k in range(32):
        m = (k + iota) & 31
        mtab[k, :] = m
        ftab[k, :] = (m >> 3) * 8192 + (m & 7) * 128 + iota
    for bc_l in range(8):
        for l0 in range(8):
            jtab[bc_l * 8 + l0, :] = bc_l * 128 + l0 * 16 + iota
    rows = (rows0, rows1)
    sg = (sg0, sg1)
    so = (so0, so1)
    pltpu.sync_copy(idx_hbm.at[wid], idx_v)

    def gather_copy(g, b):
        return pltpu.make_async_copy(
            table_hbm.at[idx_v.at[g]], rows[b], sg[b])

    def out_copies(g, b):
        gid = wid * NG + g
        base = (gid // GPF) * 524288 + (gid % GPF) * 8192
        return [pltpu.make_async_copy(
            dbuf.at[pl.ds(e4 * 8192, 8192)],
            out_hbm.at[pl.ds(base + e4 * 131072, 8192)], so[b])
            for e4 in range(4)]

    def transpose(b):
        @plsc.parallel_loop(0, 32, unroll=1)
        def kbody(k):
            m = mtab[k, :]
            ft = ftab[k, :]
            for bc_l in range(8):
                for l0 in range(8):
                    jv = jtab[bc_l * 8 + l0, :]
                    v = plsc.load_gather(rows[b], [jv, m])
                    plsc.store_scatter(dbuf, [ft + (bc_l * 1024 + l0 * 16)], v)

    def step(g, b):
        gather_copy(g, b).wait()

        @pl.when(g + 1 < NG)
        def _():
            gather_copy(g + 1, 1 - b).start()
        @pl.when(g >= 1)
        def _():
            for cp in out_copies(g - 1, 1 - b):
                cp.wait()
        transpose(b)
        for cp in out_copies(g, b):
            cp.start()

    gather_copy(0, 0).start()

    def body(j, _):
        step(2 * j, 0)
        step(2 * j + 1, 1)
        return 0

    lax.fori_loop(0, NG // 2, body, 0)
    step(NG - 1, 0)
    for cp in out_copies(NG - 1, 0):
        cp.wait()


def kernel(x, table):
    tail = table[INP - 64:].reshape(-1)
    tlin = _tpose(table.T, tail)
    tbl = tlin.reshape(INP, EMBED_DIM)
    idx = _fmt(x.T.astype(jnp.int32)).reshape(NW, NG, G)
    out = _embed(tbl, idx)
    out = out.reshape(FIELDS, 4, 128, 8, 128).transpose(2, 4, 0, 1, 3)
    return out.reshape(BATCH, FIELDS, EMBED_DIM)


# speedup vs baseline: 1.8138x; 1.0014x over previous
import functools

import jax
import jax.numpy as jnp
from jax import lax
from jax.experimental import pallas as pl
from jax.experimental.pallas import tpu as pltpu
from jax.experimental.pallas import tpu_sc as plsc

INP = 1000000
EMBED_DIM = 32
BATCH = 16384
FIELDS = 26
B = BATCH * FIELDS

NC, NS = 2, 16
NW = NC * NS
BPW = B // NW
CH = 1664
NCH = BPW // CH
COLS = BATCH // NW

_mesh = plsc.VectorSubcoreMesh(core_axis_name="c", subcore_axis_name="s")


@functools.partial(
    pl.kernel,
    out_type=jax.ShapeDtypeStruct((B,), jnp.int32),
    mesh=_mesh,
    scratch_types=[
        pltpu.VMEM((FIELDS, COLS), jnp.int32),
        pltpu.SemaphoreType.DMA,
    ],
    compiler_params=pltpu.CompilerParams(
        use_tc_tiling_on_sc=True, needs_layout_passes=False),
)
def _fmt(xt_hbm, out_hbm, buf, sem):
    wid = lax.axis_index("s") * NC + lax.axis_index("c")
    c0 = wid * COLS
    pltpu.sync_copy(xt_hbm.at[:, pl.ds(c0, COLS)], buf)
    for f in range(FIELDS):
        pltpu.async_copy(
            buf.at[f], out_hbm.at[pl.ds(f * BATCH + c0, COLS)], sem)
    for f in range(FIELDS):
        pltpu.make_async_copy(
            buf.at[f], out_hbm.at[pl.ds(f * BATCH + c0, COLS)], sem).wait()


NBLK = INP // 128


@functools.partial(
    pl.kernel,
    out_type=jax.ShapeDtypeStruct((INP * EMBED_DIM,), jnp.float32),
    mesh=_mesh,
    scratch_types=[
        pltpu.VMEM((EMBED_DIM, 128), jnp.float32),
        pltpu.VMEM((EMBED_DIM, 128), jnp.float32),
        pltpu.VMEM((4096,), jnp.float32),
        pltpu.VMEM((4096,), jnp.float32),
        pltpu.VMEM((32, 16), jnp.int32),
        pltpu.VMEM((32, 16), jnp.int32),
        pltpu.SemaphoreType.DMA,
        pltpu.SemaphoreType.DMA,
        pltpu.SemaphoreType.DMA,
        pltpu.SemaphoreType.DMA,
    ],
    compiler_params=pltpu.CompilerParams(
        use_tc_tiling_on_sc=True, needs_layout_passes=False),
)
def _tpose(tt_hbm, tail_hbm, out_hbm, buf0, buf1, dbuf0, dbuf1, mtab, dtab,
           si0, si1, so0, so1):
    wid = lax.axis_index("s") * NC + lax.axis_index("c")
    iota = lax.iota(jnp.int32, 16)
    for k in range(32):
        m = (k + iota) & 31
        mtab[k, :] = m
        dtab[k, :] = iota * 32 + m
    lvecs = [l0 * 16 + iota for l0 in range(8)]
    bufs = (buf0, buf1)
    dbufs = (dbuf0, dbuf1)
    si = (si0, si1)
    so = (so0, so1)
    npw = NBLK // NW
    lo = wid * npw

    def in_copy(c, b):
        return [pltpu.make_async_copy(
            tt_hbm.at[pl.ds(e4 * 8, 8), pl.ds(c * 128, 128)],
            bufs[b].at[pl.ds(e4 * 8, 8)], si[b]) for e4 in range(4)]

    def out_copy(c, b):
        return pltpu.make_async_copy(
            dbufs[b], out_hbm.at[pl.ds(c * 4096, 4096)], so[b])

    def transpose_block(b, ncol16):
        @plsc.parallel_loop(0, 32, unroll=2)
        def kbody(k):
            m = mtab[k, :]
            d = dtab[k, :]
            for l0 in range(ncol16):
                v = plsc.load_gather(bufs[b], [m, lvecs[l0]])
                plsc.store_scatter(dbufs[b], [d + l0 * 512], v)

    for cp in in_copy(lo, 0):
        cp.start()

    def body(j, _):
        for b in range(2):
            c = lo + 2 * j + b
            for cp in in_copy(c, b):
                cp.wait()

            @pl.when(2 * j + b + 1 < npw)
            def _():
                for cp in in_copy(c + 1, 1 - b):
                    cp.start()
            @pl.when(2 * j + b >= 2)
            def _():
                out_copy(c - 2, b).wait()
            transpose_block(b, 8)
            out_copy(c, b).start()
        return 0

    lax.fori_loop(0, npw // 2, body, 0)
    out_copy(lo + npw - 2, (npw - 2) % 2).wait()
    out_copy(lo + npw - 1, (npw - 1) % 2).wait()

    @pl.when(wid >= NW - 4)
    def _():
        c = NW * npw + (wid - (NW - 4))
        for cp in in_copy(c, 0):
            cp.start()
        for cp in in_copy(c, 0):
            cp.wait()
        transpose_block(0, 8)
        out_copy(c, 0).start()
        out_copy(c, 0).wait()

    @pl.when(wid == NW - 1)
    def _():
        pltpu.sync_copy(tail_hbm, dbuf0.at[pl.ds(0, 2048)])
        pltpu.sync_copy(dbuf0.at[pl.ds(0, 2048)],
                        out_hbm.at[pl.ds(NBLK * 4096, 2048)])


G = 1024
NG = BPW // G
GPF = BATCH // (G // 128 * 128)
OUTW = FIELDS * BATCH * EMBED_DIM


@functools.partial(
    pl.kernel,
    out_type=jax.ShapeDtypeStruct((OUTW,), jnp.float32),
    mesh=_mesh,
    scratch_types=[
        pltpu.VMEM((NG, G), jnp.int32),
        pltpu.VMEM((G, EMBED_DIM), jnp.float32),
        pltpu.VMEM((G, EMBED_DIM), jnp.float32),
        pltpu.VMEM((32 * G,), jnp.float32),
        pltpu.VMEM((32, 16), jnp.int32),
        pltpu.VMEM((32, 16), jnp.int32),
        pltpu.VMEM((64, 16), jnp.int32),
        pltpu.SemaphoreType.DMA,
        pltpu.SemaphoreType.DMA,
        pltpu.SemaphoreType.DMA,
        pltpu.SemaphoreType.DMA,
        pltpu.SemaphoreType.DMA,
    ],
    compiler_params=pltpu.CompilerParams(
        use_tc_tiling_on_sc=False, needs_layout_passes=False),
)
def _embed(table_hbm, idx_hbm, out_hbm, idx_v, rows0, rows1, dbuf,
           mtab, ftab, jtab, si, sg0, sg1, so0, so1):
    wid = lax.axis_index("s") * NC + lax.axis_index("c")
    iota = lax.iota(jnp.int32, 16)
    for k in range(32):
        m = (k + iota) & 31
        mtab[k, :] = m
        ftab[k, :] = (m >> 3) * 8192 + (m & 7) * 128 + iota
    for bc_l in range(8):
        for l0 in range(8):
            jtab[bc_l * 8 + l0, :] = bc_l * 128 + l0 * 16 + iota
    rows = (rows0, rows1)
    sg = (sg0, sg1)
    so = (so0, so1)
    pltpu.sync_copy(idx_hbm.at[wid], idx_v)

    def gather_copy(g, b):
        return pltpu.make_async_copy(
            table_hbm.at[idx_v.at[g]], rows[b], sg[b])

    def out_copies(g, b):
        gid = wid * NG + g
        base = (gid // GPF) * 524288 + (gid % GPF) * 8192
        return [pltpu.make_async_copy(
            dbuf.at[pl.ds(e4 * 8192, 8192)],
            out_hbm.at[pl.ds(base + e4 * 131072, 8192)], so[b])
            for e4 in range(4)]

    def transpose(b):
        @plsc.parallel_loop(0, 32, unroll=1)
        def kbody(k):
            m = mtab[k, :]
            ft = ftab[k, :]
            for bc_l in range(8):
                for l0 in range(8):
                    jv = jtab[bc_l * 8 + l0, :]
                    v = plsc.load_gather(rows[b], [jv, m])
                    plsc.store_scatter(dbuf, [ft + (bc_l * 1024 + l0 * 16)], v)

    def step(g, b):
        gather_copy(g, b).wait()

        @pl.when(g + 1 < NG)
        def _():
            gather_copy(g + 1, 1 - b).start()
        @pl.when(g >= 1)
        def _():
            for cp in out_copies(g - 1, 1 - b):
                cp.wait()
        transpose(b)
        for cp in out_copies(g, b):
            cp.start()

    gather_copy(0, 0).start()

    def body(j, _):
        step(2 * j, 0)
        step(2 * j + 1, 1)
        return 0

    lax.fori_loop(0, NG // 2, body, 0)
    step(NG - 1, 0)
    for cp in out_copies(NG - 1, 0):
        cp.wait()


def kernel(x, table):
    tail = table[INP - 64:].reshape(-1)
    tlin = _tpose(table.T, tail)
    tbl = tlin.reshape(INP, EMBED_DIM)
    idx = _fmt(x.T.astype(jnp.int32)).reshape(NW, NG, G)
    out = _embed(tbl, idx)
    out = out.reshape(FIELDS, 4, 128, 8, 128).transpose(2, 4, 0, 1, 3)
    return out.reshape(BATCH, FIELDS, EMBED_DIM)
